# baseline (speedup 1.0000x reference)
import jax
import jax.numpy as jnp
from jax.experimental import pallas as pl
from jax.experimental.pallas import tpu as pltpu

N_EXP = 8
TOKEN_BLOCK = 1024


def _gate_body(x_ref, wt_ref, w_out_ref, i_out_ref):
    x = x_ref[...]
    wt = wt_ref[...]
    scores = jax.lax.dot_general(
        x, wt, (((1,), (0,)), ((), ())),
        preferred_element_type=jnp.float32)
    m = jnp.max(scores, axis=1, keepdims=True)
    e = jnp.exp(scores - m)
    p = e / jnp.sum(e, axis=1, keepdims=True)

    ii = jax.lax.broadcasted_iota(jnp.int32, p.shape, 1)
    m1 = jnp.max(p, axis=1, keepdims=True)
    idx1 = jnp.min(jnp.where(p == m1, ii, N_EXP), axis=1, keepdims=True)
    p2 = jnp.where(ii == idx1, -1.0, p)
    m2 = jnp.max(p2, axis=1, keepdims=True)
    idx2 = jnp.min(jnp.where(p2 == m2, ii, N_EXP), axis=1, keepdims=True)

    w_out_ref[...] = jnp.concatenate([m1, m2], axis=1)
    i_out_ref[...] = jnp.concatenate([idx1, idx2], axis=1)


def kernel(x, weight):
    n_tokens, dim = x.shape
    grid = (n_tokens // TOKEN_BLOCK,)
    weights, indices = pl.pallas_call(
        _gate_body,
        grid=grid,
        in_specs=[
            pl.BlockSpec((TOKEN_BLOCK, dim), lambda i: (i, 0)),
            pl.BlockSpec((dim, N_EXP), lambda i: (0, 0)),
        ],
        out_specs=[
            pl.BlockSpec((TOKEN_BLOCK, 2), lambda i: (i, 0)),
            pl.BlockSpec((TOKEN_BLOCK, 2), lambda i: (i, 0)),
        ],
        out_shape=[
            jax.ShapeDtypeStruct((n_tokens, 2), jnp.float32),
            jax.ShapeDtypeStruct((n_tokens, 2), jnp.int32),
        ],
        compiler_params=pltpu.CompilerParams(
            dimension_semantics=("arbitrary",),
        ),
    )(x, weight.T)
    return weights, indices


# speedup vs baseline: 1.0194x; 1.0194x over previous
import functools

import jax
import jax.numpy as jnp
from jax import lax
from jax.experimental import pallas as pl
from jax.experimental.pallas import tpu as pltpu
from jax.experimental.pallas import tpu_sc as plsc

N_EXP = 8
TOKEN_BLOCK = 2048
LANES = 16


def _score_body(x_ref, wt_ref, s_out_ref):
    x = x_ref[...]
    w = wt_ref[...]
    s_out_ref[...] = jax.lax.dot_general(
        w, x, (((1,), (1,)), ((), ())),
        preferred_element_type=jnp.float32)


def _scores_t(x, weight):
    n_tokens, dim = x.shape
    grid = (n_tokens // TOKEN_BLOCK,)
    return pl.pallas_call(
        _score_body,
        grid=grid,
        in_specs=[
            pl.BlockSpec((TOKEN_BLOCK, dim), lambda i: (i, 0)),
            pl.BlockSpec((N_EXP, dim), lambda i: (0, 0)),
        ],
        out_specs=pl.BlockSpec((N_EXP, TOKEN_BLOCK), lambda i: (0, i)),
        out_shape=jax.ShapeDtypeStruct((N_EXP, n_tokens), jnp.float32),
        compiler_params=pltpu.CompilerParams(
            dimension_semantics=("arbitrary",),
        ),
    )(x, weight)


def _route_tec(s_hbm, w1_hbm, w2_hbm, i1_hbm, i2_hbm,
               s_v, w1_v, w2_v, i1_v, i2_v, *, tok_per_w):
    nc = 2
    wid = lax.axis_index("s") * nc + lax.axis_index("c")
    base = wid * tok_per_w
    pltpu.sync_copy(s_hbm.at[:, pl.ds(base, tok_per_w)], s_v)

    def step(i, _):
        off = i * LANES
        s = [s_v[e, pl.ds(off, LANES)] for e in range(N_EXP)]
        m = s[0]
        for e in range(1, N_EXP):
            m = jnp.maximum(m, s[e])
        ex = [jnp.exp(s[e] - m) for e in range(N_EXP)]
        tot = ex[0]
        for e in range(1, N_EXP):
            tot = tot + ex[e]
        p = [ex[e] / tot for e in range(N_EXP)]

        m1 = p[0]
        for e in range(1, N_EXP):
            m1 = jnp.maximum(m1, p[e])
        idx1 = jnp.full((LANES,), 0, jnp.int32)
        for e in range(N_EXP - 1, -1, -1):
            idx1 = jnp.where(p[e] == m1, jnp.full((LANES,), e, jnp.int32),
                             idx1)
        neg = jnp.full((LANES,), -1.0, jnp.float32)
        p2 = [jnp.where(idx1 == e, neg, p[e]) for e in range(N_EXP)]
        m2 = p2[0]
        for e in range(1, N_EXP):
            m2 = jnp.maximum(m2, p2[e])
        idx2 = jnp.full((LANES,), 0, jnp.int32)
        for e in range(N_EXP - 1, -1, -1):
            idx2 = jnp.where(p2[e] == m2, jnp.full((LANES,), e, jnp.int32),
                             idx2)

        w1_v[pl.ds(off, LANES)] = m1
        w2_v[pl.ds(off, LANES)] = m2
        i1_v[pl.ds(off, LANES)] = idx1
        i2_v[pl.ds(off, LANES)] = idx2
        return _

    lax.fori_loop(0, tok_per_w // LANES, step, 0)
    pltpu.sync_copy(w1_v, w1_hbm.at[pl.ds(base, tok_per_w)])
    pltpu.sync_copy(w2_v, w2_hbm.at[pl.ds(base, tok_per_w)])
    pltpu.sync_copy(i1_v, i1_hbm.at[pl.ds(base, tok_per_w)])
    pltpu.sync_copy(i2_v, i2_hbm.at[pl.ds(base, tok_per_w)])


def _route_sc(scores_t):
    n_exp, n_tokens = scores_t.shape
    n_cores, n_subcores = 2, 16
    tok_per_w = n_tokens // (n_cores * n_subcores)
    mesh = plsc.VectorSubcoreMesh(
        core_axis_name="c", subcore_axis_name="s",
        num_cores=n_cores, num_subcores=n_subcores)
    k = pl.kernel(
        functools.partial(_route_tec, tok_per_w=tok_per_w),
        out_type=[
            jax.ShapeDtypeStruct((n_tokens,), jnp.float32),
            jax.ShapeDtypeStruct((n_tokens,), jnp.float32),
            jax.ShapeDtypeStruct((n_tokens,), jnp.int32),
            jax.ShapeDtypeStruct((n_tokens,), jnp.int32),
        ],
        mesh=mesh,
        scratch_types=[
            pltpu.VMEM((n_exp, tok_per_w), jnp.float32),
            pltpu.VMEM((tok_per_w,), jnp.float32),
            pltpu.VMEM((tok_per_w,), jnp.float32),
            pltpu.VMEM((tok_per_w,), jnp.int32),
            pltpu.VMEM((tok_per_w,), jnp.int32),
        ],
    )
    w1, w2, i1, i2 = k(scores_t)
    return (jnp.stack([w1, w2], axis=1), jnp.stack([i1, i2], axis=1))


def kernel(x, weight):
    scores_t = _scores_t(x, weight)
    weights, indices = _route_sc(scores_t)
    return weights, indices


# speedup vs baseline: 1.0396x; 1.0198x over previous
import jax
import jax.numpy as jnp
from jax.experimental import pallas as pl
from jax.experimental.pallas import tpu as pltpu

N_EXP = 8
TOKEN_BLOCK = 2048
HALF = TOKEN_BLOCK // 2


def _gate_half(x, wt):
    scores = jax.lax.dot_general(
        x, wt, (((1,), (0,)), ((), ())),
        preferred_element_type=jnp.float32)
    m = jnp.max(scores, axis=1, keepdims=True)
    e = jnp.exp(scores - m)
    p = e / jnp.sum(e, axis=1, keepdims=True)

    ii = jax.lax.broadcasted_iota(jnp.int32, p.shape, 1)
    m1 = jnp.max(p, axis=1, keepdims=True)
    idx1 = jnp.min(jnp.where(p == m1, ii, N_EXP), axis=1, keepdims=True)
    p2 = jnp.where(ii == idx1, -1.0, p)
    m2 = jnp.max(p2, axis=1, keepdims=True)
    idx2 = jnp.min(jnp.where(p2 == m2, ii, N_EXP), axis=1, keepdims=True)
    return (jnp.concatenate([m1, m2], axis=1),
            jnp.concatenate([idx1, idx2], axis=1))


def _gate_body(xa_ref, xb_ref, wt_ref, w_out_ref, i_out_ref):
    wt = wt_ref[...]
    wa, ia = _gate_half(xa_ref[...], wt)
    wb, ib = _gate_half(xb_ref[...], wt)
    w_out_ref[0:HALF, :] = wa
    w_out_ref[HALF:TOKEN_BLOCK, :] = wb
    i_out_ref[0:HALF, :] = ia
    i_out_ref[HALF:TOKEN_BLOCK, :] = ib


def kernel(x, weight):
    n_tokens, dim = x.shape
    grid = (n_tokens // TOKEN_BLOCK,)
    weights, indices = pl.pallas_call(
        _gate_body,
        grid=grid,
        in_specs=[
            pl.BlockSpec((HALF, dim), lambda i: (2 * i, 0)),
            pl.BlockSpec((HALF, dim), lambda i: (2 * i + 1, 0)),
            pl.BlockSpec((dim, N_EXP), lambda i: (0, 0)),
        ],
        out_specs=[
            pl.BlockSpec((TOKEN_BLOCK, 2), lambda i: (i, 0)),
            pl.BlockSpec((TOKEN_BLOCK, 2), lambda i: (i, 0)),
        ],
        out_shape=[
            jax.ShapeDtypeStruct((n_tokens, 2), jnp.float32),
            jax.ShapeDtypeStruct((n_tokens, 2), jnp.int32),
        ],
        compiler_params=pltpu.CompilerParams(
            dimension_semantics=("arbitrary",),
        ),
    )(x, x, weight.T)
    return weights, indices
